# baseline (speedup 1.0000x reference)
import functools

import jax
import jax.numpy as jnp
from jax import lax
from jax.experimental import pallas as pl
from jax.experimental.pallas import tpu as pltpu
from jax.experimental.pallas import tpu_sc as plsc

_D = 64
_SCALE = 8.0
_NC, _NS = 2, 16
_NW = _NC * _NS
_CHUNK = 128
_K = 4
_SUP = _CHUNK * _K


@functools.lru_cache(maxsize=None)
def _make_emb(nh, nb):
    assert nb == _SUP * _NW and nh % 2 == 0
    nit = nb // _CHUNK
    mesh = plsc.VectorSubcoreMesh(core_axis_name="c", subcore_axis_name="s")

    @functools.partial(
        pl.kernel,
        out_type=jax.ShapeDtypeStruct((nh, _D // 8, nit, 8, _CHUNK), jnp.float32),
        mesh=mesh,
        compiler_params=pltpu.CompilerParams(
            use_tc_tiling_on_sc=False, needs_layout_passes=False
        ),
        scratch_types=[
            pltpu.VMEM((2, _K, _CHUNK), jnp.int32),
            pltpu.VMEM((2, _SUP, _D), jnp.float32),
            pltpu.VMEM((_D // 8, _K, 8, _CHUNK), jnp.float32),
            pltpu.SemaphoreType.DMA,
            pltpu.SemaphoreType.DMA,
        ],
    )
    def emb(idx_hbm, table_hbm, out_hbm, idx_v, rows_v, t_v, sem0, sem1):
        wid = lax.axis_index("s") * _NC + lax.axis_index("c")
        it0 = wid * _K

        def fire(h, b, sem):
            pltpu.sync_copy(idx_hbm.at[h, pl.ds(it0, _K)], idx_v.at[b])
            for j in range(_K):
                pltpu.async_copy(
                    table_hbm.at[idx_v.at[b, j]],
                    rows_v.at[b, pl.ds(j * _CHUNK, _CHUNK)],
                    sem,
                )

        def drain(sem, b):
            pltpu.make_async_copy(
                table_hbm.at[pl.ds(0, _SUP)], rows_v.at[b], sem
            ).wait()

        def scale_store(h, b):
            buf = rows_v.at[b]

            for jt in range(_K):

                @plsc.parallel_loop(0, _CHUNK, step=16, unroll=1)
                def _transpose(ic, jt=jt):
                    rows = lax.iota(jnp.int32, 16) + (jt * _CHUNK + ic)
                    for k in range(_D):
                        v = plsc.load_gather(
                            buf, [rows, jnp.full((16,), k, jnp.int32)]
                        )
                        t_v[k // 8, jt, k % 8, pl.ds(ic, 16)] = v * _SCALE

            pltpu.sync_copy(t_v, out_hbm.at[h, :, pl.ds(it0, _K)])

        fire(0, 0, sem0)

        def body(p, carry):
            h0 = 2 * p
            fire(h0 + 1, 1, sem1)
            drain(sem0, 0)
            scale_store(h0, 0)
            fire(jnp.minimum(h0 + 2, nh - 1), 0, sem0)
            drain(sem1, 1)
            scale_store(h0 + 1, 1)
            return carry

        lax.fori_loop(0, nh // 2, body, 0)
        drain(sem0, 0)

    return emb


def kernel(x, table):
    nb, nh = x.shape
    nit = nb // _CHUNK
    xt = x.T.reshape(nh, nit, _CHUNK).astype(jnp.int32)
    out5 = _make_emb(nh, nb)(xt, table)
    return jnp.transpose(out5, (2, 4, 0, 1, 3)).reshape(nb, nh, _D)
